# baseline (speedup 1.0000x reference)
import functools

import jax
import jax.numpy as jnp
import numpy as np
from jax.experimental import pallas as pl
from jax.experimental.pallas import tpu as pltpu

_K = 100
_RUN = 128
_CB = 4000
_SLOTS = 4096
_DEPTH = 256
_WIDTH = 16
_NEG = float(np.finfo(np.float32).min)


def _cx_desc(A, s):
    P = A.shape[0] // (2 * s)
    A5 = A.reshape(P, 2, s, *A.shape[1:])
    a = A5[:, 0]
    b = A5[:, 1]
    return jnp.stack([jnp.maximum(a, b), jnp.minimum(a, b)], axis=1).reshape(A.shape)


def _dir_sign(m, dtype):
    i = jax.lax.broadcasted_iota(jnp.int32, (_DEPTH, 1, 1), 0)
    d = ((i >> 7) ^ ((i & 127) // m)) & 1
    return (1 - 2 * d).astype(dtype)


def _merge128_desc(M):
    for s in (64, 32, 16, 8, 4, 2, 1):
        M = _cx_desc(M, s)
    return M


def _colsign(Wd, ndesc, dtype):
    w = jax.lax.broadcasted_iota(jnp.int32, (1, Wd, 1), 1)
    return jnp.where(w < ndesc, 1, -1).astype(dtype)


def _halve(M, ndesc):
    Wd = M.shape[1]
    M = jnp.concatenate([M[:, : Wd // 2], M[:, Wd // 2 :]], axis=0)
    M = jnp.maximum(M[:_RUN], -M[_RUN:])
    if ndesc == 0:
        M = -M
    else:
        M = M * _colsign(Wd // 2, ndesc, M.dtype)
    return _merge128_desc(M)


def _batch_top128(scores):
    Q = scores.shape[1]
    A = scores.reshape(_DEPTH, _WIDTH, Q)
    A = A * _dir_sign(2, A.dtype)
    for m in (2, 4, 8, 16, 32, 64, 128):
        s = m // 2
        while s >= 1:
            A = _cx_desc(A, s)
            s //= 2
        if m < 128:
            A = A * (_dir_sign(m, A.dtype) * _dir_sign(2 * m, A.dtype))
    M = jnp.maximum(A[:_RUN], -A[_RUN:])
    M = M * _colsign(_WIDTH, 8, M.dtype)
    M = _merge128_desc(M)
    M = _halve(M, 4)
    M = _halve(M, 2)
    M = _halve(M, 1)
    M = _halve(M, 0)
    return M.reshape(_RUN, Q)


def _body(n_blocks, c_ref, qt_ref, o_ref, state_ref):
    i = pl.program_id(0)

    @pl.when(i == 0)
    def _init():
        state_ref[...] = jnp.full(state_ref.shape, _NEG, jnp.bfloat16)

    q = qt_ref[...]
    scores = jnp.dot(c_ref[...], q, preferred_element_type=jnp.float32)
    scores = scores.astype(jnp.bfloat16)
    pad = jnp.full((_SLOTS - _CB, scores.shape[1]), _NEG, jnp.bfloat16)
    scores = jnp.concatenate([scores, pad], axis=0)
    btop_neg = _batch_top128(scores)
    merged = jnp.maximum(state_ref[...], -btop_neg)
    merged = _merge128_desc(merged[:, None, :]).reshape(btop_neg.shape)
    state_ref[...] = merged

    @pl.when(i == n_blocks - 1)
    def _out():
        o_ref[...] = merged


@jax.jit
def kernel(query_embeddings, candidates):
    nq, d = query_embeddings.shape
    n_blocks = candidates.shape[0] // _CB
    qt = query_embeddings.T.astype(jnp.bfloat16)
    candidates = candidates.astype(jnp.bfloat16)
    out = pl.pallas_call(
        functools.partial(_body, n_blocks),
        grid=(n_blocks,),
        in_specs=[
            pl.BlockSpec((_CB, d), lambda i: (i, 0)),
            pl.BlockSpec((d, nq), lambda i: (0, 0)),
        ],
        out_specs=pl.BlockSpec((_RUN, nq), lambda i: (0, 0)),
        out_shape=jax.ShapeDtypeStruct((_RUN, nq), jnp.bfloat16),
        scratch_shapes=[pltpu.VMEM((_RUN, nq), jnp.bfloat16)],
        compiler_params=pltpu.CompilerParams(
            dimension_semantics=("arbitrary",),
        ),
    )(candidates, qt)
    return out[:_K].T.astype(jnp.float32)


# speedup vs baseline: 1.3943x; 1.3943x over previous
import functools

import jax
import jax.numpy as jnp
import numpy as np
from jax.experimental import pallas as pl
from jax.experimental.pallas import tpu as pltpu

_K = 100
_RUN = 128
_CB = 4000
_SLOTS = 4096
_DEPTH = 256
_WIDTH = 16
_NEG = float(np.finfo(np.float32).min)


def _cx_desc(A, s):
    P = A.shape[0] // (2 * s)
    A5 = A.reshape(P, 2, s, *A.shape[1:])
    a = A5[:, 0]
    b = A5[:, 1]
    return jnp.stack([jnp.maximum(a, b), jnp.minimum(a, b)], axis=1).reshape(A.shape)


def _dir_sign(m, dtype):
    i = jax.lax.broadcasted_iota(jnp.int32, (_DEPTH, 1, 1), 0)
    d = ((i >> 7) ^ ((i & 127) // m)) & 1
    return (1 - 2 * d).astype(dtype)


def _merge128_desc(M):
    for s in (64, 32, 16, 8, 4, 2, 1):
        M = _cx_desc(M, s)
    return M


def _colsign(Wd, ndesc, dtype):
    w = jax.lax.broadcasted_iota(jnp.int32, (1, Wd, 1), 1)
    return jnp.where(w < ndesc, 1, -1).astype(dtype)


def _fold_halve(M, last=False):
    q = M.shape[2]
    half = M.shape[1] // 2
    M = jnp.concatenate([M[:, :half], M[:, half:]], axis=0)
    M = jnp.maximum(M[:_RUN], -M[_RUN:])
    if q // 2 >= 128:
        M = M.reshape(_RUN, 2 * half, q // 2)
    if last:
        M = -M
    else:
        M = M * _colsign(M.shape[1], M.shape[1] // 2, M.dtype)
    return _merge128_desc(M)


def _batch_top128(scores):
    Q = scores.shape[1]
    A = scores.reshape(_DEPTH, _WIDTH, Q)
    A = A * _dir_sign(2, A.dtype)
    for m in (2, 4, 8, 16, 32, 64, 128):
        s = m // 2
        while s >= 1:
            A = _cx_desc(A, s)
            s //= 2
        if m < 128:
            A = A * (_dir_sign(m, A.dtype) * _dir_sign(2 * m, A.dtype))
    M = jnp.maximum(A[:_RUN], -A[_RUN:])
    M = M * _colsign(_WIDTH, 8, M.dtype)
    M = _merge128_desc(M)
    M = _fold_halve(M)
    M = _fold_halve(M)
    M = _fold_halve(M)
    M = _fold_halve(M, last=True)
    return M


def _body(n_blocks, c_ref, qt_ref, o_ref, state_ref):
    i = pl.program_id(0)

    @pl.when(i == 0)
    def _init():
        state_ref[...] = jnp.full(state_ref.shape, _NEG, jnp.bfloat16)

    q = qt_ref[...]
    scores = jnp.dot(c_ref[...], q, preferred_element_type=jnp.float32)
    scores = scores.astype(jnp.bfloat16)
    pad = jnp.full((_SLOTS - _CB, scores.shape[1]), _NEG, jnp.bfloat16)
    scores = jnp.concatenate([scores, pad], axis=0)
    btop_neg = _batch_top128(scores)
    merged = jnp.maximum(state_ref[...], -btop_neg)
    merged = _merge128_desc(merged)
    state_ref[...] = merged

    @pl.when(i == n_blocks - 1)
    def _out():
        o_ref[...] = merged


@jax.jit
def kernel(query_embeddings, candidates):
    nq, d = query_embeddings.shape
    n_blocks = candidates.shape[0] // _CB
    qt = query_embeddings.T
    out = pl.pallas_call(
        functools.partial(_body, n_blocks),
        grid=(n_blocks,),
        in_specs=[
            pl.BlockSpec((_CB, d), lambda i: (i, 0)),
            pl.BlockSpec((d, nq), lambda i: (0, 0)),
        ],
        out_specs=pl.BlockSpec((_RUN, 8, nq // 8), lambda i: (0, 0, 0)),
        out_shape=jax.ShapeDtypeStruct((_RUN, 8, nq // 8), jnp.bfloat16),
        scratch_shapes=[pltpu.VMEM((_RUN, 8, nq // 8), jnp.bfloat16)],
        compiler_params=pltpu.CompilerParams(
            dimension_semantics=("arbitrary",),
        ),
    )(candidates, qt)
    return out.reshape(_RUN, nq)[:_K].T.astype(jnp.float32)
